# Initial kernel scaffold; baseline (speedup 1.0000x reference)
import jax
import jax.numpy as jnp
from jax.experimental import pallas as pl


def kernel(ctrl_tokens, embed_table, W, b):
    raise NotImplementedError("write your pallas kernel here")


# speedup vs baseline: 3.2558x; 3.2558x over previous
import functools

import jax
import jax.numpy as jnp
from jax import lax
from jax.experimental import pallas as pl
from jax.experimental.pallas import tpu as pltpu
from jax.experimental.pallas import tpu_sc as plsc

D_MODEL = 128


def _build_sc_gather(V, E, N):
    info = plsc.get_sparse_core_info()
    NC, NS = info.num_cores, info.num_subcores
    NW = NC * NS
    assert N % (8 * NW) == 0
    n_per_w = N // NW
    mesh = plsc.VectorSubcoreMesh(core_axis_name="c", subcore_axis_name="s")

    @functools.partial(
        pl.kernel,
        out_type=jax.ShapeDtypeStruct((N, E), jnp.float32),
        mesh=mesh,
        compiler_params=pltpu.CompilerParams(use_tc_tiling_on_sc=False),
        scratch_types=[
            pltpu.VMEM((n_per_w,), jnp.int32),
            pltpu.VMEM((n_per_w, E), jnp.float32),
            pltpu.SemaphoreType.DMA,
        ],
    )
    def gather_kernel(table_hbm, idx_hbm, out_hbm, idx_v, rows_v, sem):
        wid = lax.axis_index("s") * NC + lax.axis_index("c")
        base = wid * n_per_w
        pltpu.sync_copy(idx_hbm.at[pl.ds(base, n_per_w)], idx_v)
        pltpu.async_copy(table_hbm.at[idx_v], rows_v, sem).wait()
        pltpu.sync_copy(rows_v, out_hbm.at[pl.ds(base, n_per_w)])

    return gather_kernel


def _mm_body(e_ref, w_ref, b_ref, o_ref):
    o_ref[...] = lax.dot_general(
        e_ref[...], w_ref[...],
        dimension_numbers=(((1,), (1,)), ((), ())),
        preferred_element_type=jnp.float32,
    ) + b_ref[...]


def _tc_project(e, W, b2d, block_m):
    B = e.shape[0]
    return pl.pallas_call(
        _mm_body,
        out_shape=jax.ShapeDtypeStruct((B, D_MODEL), jnp.float32),
        grid=(B // block_m,),
        in_specs=[
            pl.BlockSpec((block_m, D_MODEL), lambda i: (i, 0)),
            pl.BlockSpec((D_MODEL, D_MODEL), lambda i: (0, 0)),
            pl.BlockSpec((1, D_MODEL), lambda i: (0, 0)),
        ],
        out_specs=pl.BlockSpec((block_m, D_MODEL), lambda i: (i, 0)),
    )(e, W, b2d)


def kernel(ctrl_tokens, embed_table, W, b):
    B, S = ctrl_tokens.shape
    V, E = embed_table.shape
    N = B * S
    idx = ctrl_tokens.reshape(N).astype(jnp.int32)
    rows = _build_sc_gather(V, E, N)(embed_table, idx)
    e = rows.reshape(B, S * E)
    out = _tc_project(e, W, b.reshape(1, D_MODEL), 2048)
    return out[..., None]


# speedup vs baseline: 3.4234x; 1.0515x over previous
import functools

import jax
import jax.numpy as jnp
from jax import lax
from jax.experimental import pallas as pl
from jax.experimental.pallas import tpu as pltpu
from jax.experimental.pallas import tpu_sc as plsc

D_MODEL = 128


def _build_sc_gather(V, E, B, S):
    info = plsc.get_sparse_core_info()
    NC, NS = info.num_cores, info.num_subcores
    NW = NC * NS
    n_groups = NW // S
    assert B % (8 * n_groups) == 0
    b_per_g = B // n_groups
    mesh = plsc.VectorSubcoreMesh(core_axis_name="c", subcore_axis_name="s")

    @functools.partial(
        pl.kernel,
        out_type=jax.ShapeDtypeStruct((B, S * E), jnp.float32),
        mesh=mesh,
        compiler_params=pltpu.CompilerParams(use_tc_tiling_on_sc=False),
        scratch_types=[
            pltpu.VMEM((b_per_g,), jnp.int32),
            pltpu.VMEM((b_per_g, E), jnp.float32),
            pltpu.SemaphoreType.DMA,
        ],
    )
    def gather_kernel(table_hbm, idx_hbm, out_hbm, idx_v, rows_v, sem):
        wid = lax.axis_index("s") * NC + lax.axis_index("c")
        s = wid % S
        base = (wid // S) * b_per_g
        pltpu.sync_copy(idx_hbm.at[pl.ds(s * B + base, b_per_g)], idx_v)
        pltpu.async_copy(table_hbm.at[idx_v], rows_v, sem).wait()
        pltpu.sync_copy(
            rows_v, out_hbm.at[pl.ds(base, b_per_g), pl.ds(s * E, E)]
        )

    return gather_kernel


def _mm_body(e_ref, w_ref, b_ref, o_ref):
    o_ref[...] = lax.dot_general(
        e_ref[...], w_ref[...],
        dimension_numbers=(((1,), (1,)), ((), ())),
        preferred_element_type=jnp.float32,
    ) + b_ref[...]


def _tc_project(e, W, b2d, block_m):
    B = e.shape[0]
    return pl.pallas_call(
        _mm_body,
        out_shape=jax.ShapeDtypeStruct((B, D_MODEL), jnp.float32),
        grid=(B // block_m,),
        in_specs=[
            pl.BlockSpec((block_m, D_MODEL), lambda i: (i, 0)),
            pl.BlockSpec((D_MODEL, D_MODEL), lambda i: (0, 0)),
            pl.BlockSpec((1, D_MODEL), lambda i: (0, 0)),
        ],
        out_specs=pl.BlockSpec((block_m, D_MODEL), lambda i: (i, 0)),
    )(e, W, b2d)


def kernel(ctrl_tokens, embed_table, W, b):
    B, S = ctrl_tokens.shape
    V, E = embed_table.shape
    idx = ctrl_tokens.T.reshape(S * B).astype(jnp.int32)
    e = _build_sc_gather(V, E, B, S)(embed_table, idx)
    out = _tc_project(e, W, b.reshape(1, D_MODEL), 2048)
    return out[..., None]
